# baseline (speedup 1.0000x reference)
import functools

import jax
import jax.numpy as jnp
from jax import lax
from jax.experimental import pallas as pl
from jax.experimental.pallas import tpu as pltpu
from jax.experimental.pallas import tpu_sc as plsc

_FACTOR = 16


@functools.lru_cache(maxsize=None)
def _build(batch):
    info = plsc.get_sparse_core_info()
    nc, ns = info.num_cores, info.num_subcores
    nw = nc * ns
    assert batch % (8 * nw) == 0
    bpw = batch // nw

    mesh = plsc.VectorSubcoreMesh(core_axis_name="c", subcore_axis_name="s")

    @functools.partial(
        pl.kernel,
        mesh=mesh,
        out_type=jax.ShapeDtypeStruct((batch,), jnp.float32),
        compiler_params=pltpu.CompilerParams(
            needs_layout_passes=False, use_tc_tiling_on_sc=False),
        scratch_types=[
            pltpu.VMEM((bpw,), jnp.int32),
            pltpu.VMEM((bpw,), jnp.int32),
            pltpu.VMEM((bpw, _FACTOR), jnp.float32),
            pltpu.VMEM((bpw, _FACTOR), jnp.float32),
            pltpu.VMEM((bpw,), jnp.float32),
            pltpu.VMEM((bpw,), jnp.float32),
            pltpu.VMEM((bpw,), jnp.float32),
            pltpu.VMEM((bpw,), jnp.float32),
            pltpu.SemaphoreType.DMA,
            pltpu.SemaphoreType.DMA,
            pltpu.SemaphoreType.DMA,
            pltpu.SemaphoreType.DMA,
        ],
    )
    def cdcf_kernel(user_hbm, item_hbm, avg_hbm, eu_hbm, ei_hbm,
                    bu_hbm, bi_hbm, out_hbm,
                    uidx_v, iidx_v, ue_v, ie_v, bu_v, bi_v, avg_v, out_v,
                    sem_ue, sem_ie, sem_bu, sem_bi):
        wid = lax.axis_index("s") * nc + lax.axis_index("c")
        base = wid * bpw

        pltpu.sync_copy(user_hbm.at[pl.ds(base, bpw)], uidx_v)
        pltpu.sync_copy(item_hbm.at[pl.ds(base, bpw)], iidx_v)

        cue = pltpu.async_copy(eu_hbm.at[uidx_v], ue_v, sem_ue)
        cie = pltpu.async_copy(ei_hbm.at[iidx_v], ie_v, sem_ie)
        cbu = pltpu.async_copy(bu_hbm.at[uidx_v], bu_v, sem_bu)
        cbi = pltpu.async_copy(bi_hbm.at[iidx_v], bi_v, sem_bi)

        pltpu.sync_copy(avg_hbm.at[pl.ds(base, bpw)], avg_v)

        cue.wait()
        cie.wait()
        cbu.wait()
        cbi.wait()

        lane = lax.iota(jnp.int32, 16)

        def grp(g, carry):
            s = pl.ds(g * 16, 16)
            rows = g * 16 + lane
            acc = avg_v[s] + bu_v[s] + bi_v[s]
            for f in range(_FACTOR):
                col = jnp.full((16,), f, jnp.int32)
                ucol = plsc.load_gather(ue_v, [rows, col])
                icol = plsc.load_gather(ie_v, [rows, col])
                acc = acc + ucol * icol
            out_v[s] = acc
            return carry

        lax.fori_loop(0, bpw // 16, grp, 0)

        pltpu.sync_copy(out_v, out_hbm.at[pl.ds(base, bpw)])

    return cdcf_kernel


def kernel(user, item, average, embed_user, embed_item, user_bias, item_bias):
    user = user.astype(jnp.int32)
    item = item.astype(jnp.int32)
    fn = _build(user.shape[0])
    return fn(user, item, average, embed_user, embed_item, user_bias, item_bias)
